# Initial kernel scaffold; baseline (speedup 1.0000x reference)
import jax
import jax.numpy as jnp
from jax.experimental import pallas as pl


def kernel(x, target):
    raise NotImplementedError("write your pallas kernel here")


# speedup vs baseline: 5.4004x; 5.4004x over previous
import functools
import math

import jax
import jax.numpy as jnp
from jax.experimental import pallas as pl

_SIZE = 32000
_N = 2048
_EPS = 0.1 / (_SIZE - 2)
_CONF = 0.9
_C_ROW = _CONF * math.log(_CONF) + 0.1 * math.log(_EPS)

_BR = 256
_BC = 3200
_R = _N // _BR
_C = _SIZE // _BC


def _loss_body(x_ref, tgt_ref, o_ref):
    r = pl.program_id(0)
    c = pl.program_id(1)
    first = jnp.logical_and(r == 0, c == 0)

    xb = x_ref[...]
    tgt = tgt_ref[0, 0, :]
    maskf = (tgt != 0).astype(jnp.float32)

    msum = jnp.sum(jnp.sum(xb, axis=1) * maskf)

    cols = c * _BC + jax.lax.broadcasted_iota(jnp.int32, (_BR, _BC), 1)
    hit = (cols == tgt[:, None]) & (maskf[:, None] > 0)
    asum = jnp.sum(jnp.where(hit, xb, 0.0))

    m0 = jnp.sum(xb[:, 0] * maskf)
    c0_term = jnp.where(c == 0, _EPS * m0 + _C_ROW * jnp.sum(maskf), 0.0)

    contrib = (-_EPS * msum - (_CONF - _EPS) * asum + c0_term).reshape(1, 1)
    o_ref[...] = jnp.where(first, contrib, o_ref[...] + contrib)


@functools.partial(jax.jit, static_argnames=())
def kernel(x, target):
    tgt3 = target.astype(jnp.int32).reshape(_R, 1, _BR)
    out = pl.pallas_call(
        _loss_body,
        grid=(_R, _C),
        in_specs=[
            pl.BlockSpec((_BR, _BC), lambda r, c: (r, c)),
            pl.BlockSpec((1, 1, _BR), lambda r, c: (r, 0, 0)),
        ],
        out_specs=pl.BlockSpec((1, 1), lambda r, c: (0, 0)),
        out_shape=jax.ShapeDtypeStruct((1, 1), jnp.float32),
    )(x, tgt3)
    return out.reshape(())
